# baseline (speedup 1.0000x reference)
import functools

import jax
import jax.numpy as jnp
from jax import lax
from jax.experimental import pallas as pl
from jax.experimental.pallas import tpu as pltpu
from jax.experimental.pallas import tpu_sc as plsc

_K = 128
_SIGMA = 0.5
_EPS = 1e-5
_H = 512


def _fps_kernel(pts_ref, tgt_ref, idx_ref, cp_ref, nn_ref):
    P = pts_ref[...]
    Bv = P.shape[0] // 3
    S, L = P.shape[1], P.shape[2]
    Nv = S * L
    n_id = (jax.lax.broadcasted_iota(jnp.int32, (Bv, S, L), 1) * L
            + jax.lax.broadcasted_iota(jnp.int32, (Bv, S, L), 2))
    kidx = jax.lax.broadcasted_iota(jnp.int32, (Bv, _K), 1)
    boff = jax.lax.broadcasted_iota(jnp.int32, (Bv, 1), 0) * Nv

    def step(k, carry):
        dist, far, idxs, cxa, cya, cza = carry
        oh = n_id == far
        ohb = jnp.concatenate([oh, oh, oh], axis=0)
        c = jnp.sum(jnp.where(ohb, P, 0.0), axis=(1, 2),
                    keepdims=True)
        sel = kidx == k
        idxs = jnp.where(sel, far.reshape(Bv, 1) + boff, idxs)
        cxa = jnp.where(sel, c[0 * Bv:1 * Bv].reshape(Bv, 1), cxa)
        cya = jnp.where(sel, c[1 * Bv:2 * Bv].reshape(Bv, 1), cya)
        cza = jnp.where(sel, c[2 * Bv:3 * Bv].reshape(Bv, 1), cza)
        D = P - c
        sq = D * D
        d = (sq[0 * Bv:1 * Bv] + sq[1 * Bv:2 * Bv]) + sq[2 * Bv:3 * Bv]
        dist = jnp.minimum(dist, d)
        m = jnp.max(dist, axis=(1, 2), keepdims=True)
        far = jnp.min(jnp.where(dist == m, n_id, Nv), axis=(1, 2),
                      keepdims=True).astype(jnp.int32)
        return dist, far, idxs, cxa, cya, cza

    dist0 = jnp.full((Bv, S, L), jnp.inf, jnp.float32)
    far0 = jnp.zeros((Bv, 1, 1), jnp.int32)
    z = jnp.zeros((Bv, _K), jnp.float32)
    carry = (dist0, far0, jnp.zeros((Bv, _K), jnp.int32), z, z, z)
    _, _, idxs, cxa, cya, cza = jax.lax.fori_loop(0, _K, step, carry,
                                                  unroll=4)
    idx_ref[...] = idxs
    cp_ref[0] = cxa
    cp_ref[1] = cya
    cp_ref[2] = cza

    Mv = tgt_ref.shape[2]
    lane = jax.lax.broadcasted_iota(jnp.int32, (_K, Mv), 1)
    for b in range(Bv):
        cx = cxa[b].reshape(_K, 1)
        cy = cya[b].reshape(_K, 1)
        cz = cza[b].reshape(_K, 1)
        tx = tgt_ref[0, b].reshape(1, Mv)
        ty = tgt_ref[1, b].reshape(1, Mv)
        tz = tgt_ref[2, b].reshape(1, Mv)
        dx = cx - tx
        d2 = dx * dx
        dy = cy - ty
        d2 = d2 + dy * dy
        dz = cz - tz
        d2 = d2 + dz * dz
        dk = jnp.sqrt(jnp.maximum(d2, 0.0))
        mk = jnp.min(dk, axis=1, keepdims=True)
        nn = jnp.min(jnp.where(dk == mk, lane, Mv), axis=1, keepdims=True)
        nn_ref[b, :] = (nn + b * Mv).reshape(_K).astype(jnp.int32)


def _fps_knn(src_t, tgt_t):
    B = src_t.shape[1]
    N = src_t.shape[2]
    S = 8
    pts = src_t.reshape(3 * B, S, N // S)
    return pl.pallas_call(
        _fps_kernel,
        out_shape=(
            jax.ShapeDtypeStruct((B, _K), jnp.int32),
            jax.ShapeDtypeStruct((3, B, _K), jnp.float32),
            jax.ShapeDtypeStruct((B, _K), jnp.int32),
        ),
    )(pts, tgt_t)


def _sc_gather2(tab1, idx1, tab2, idx2):
    R = idx1.shape[0]
    F = tab1.shape[1]
    info = plsc.get_sparse_core_info()
    nw = info.num_cores * info.num_subcores
    bpw = R // nw
    mesh = plsc.VectorSubcoreMesh(core_axis_name="c", subcore_axis_name="s")

    @functools.partial(
        pl.kernel,
        mesh=mesh,
        out_type=(
            jax.ShapeDtypeStruct((R, F), jnp.float32),
            jax.ShapeDtypeStruct((R, F), jnp.float32),
        ),
        scratch_types=[
            pltpu.VMEM((bpw,), jnp.int32),
            pltpu.VMEM((bpw, F), jnp.float32),
            pltpu.VMEM((bpw,), jnp.int32),
            pltpu.VMEM((bpw, F), jnp.float32),
            pltpu.SemaphoreType.DMA,
            pltpu.SemaphoreType.DMA,
        ],
    )
    def gather_k(t1, i1, t2, i2, o1, o2, iv1, rv1, iv2, rv2, s1, s2):
        wid = lax.axis_index("s") * info.num_cores + lax.axis_index("c")
        base = wid * bpw
        ci1 = pltpu.async_copy(i1.at[pl.ds(base, bpw)], iv1, s1)
        ci2 = pltpu.async_copy(i2.at[pl.ds(base, bpw)], iv2, s2)
        ci1.wait()
        c1 = pltpu.async_copy(t1.at[iv1], rv1, s1)
        ci2.wait()
        c2 = pltpu.async_copy(t2.at[iv2], rv2, s2)
        c1.wait()
        pltpu.sync_copy(rv1, o1.at[pl.ds(base, bpw)])
        c2.wait()
        pltpu.sync_copy(rv2, o2.at[pl.ds(base, bpw)])

    return gather_k(tab1, idx1, tab2, idx2)


def _bn(x, g, b):
    mu = jnp.mean(x, axis=0, keepdims=True)
    var = jnp.mean((x - mu) ** 2, axis=0, keepdims=True)
    return (x - mu) / jnp.sqrt(var + _EPS) * g[None, :] + b[None, :]


def _wts_kernel(src_ref, cp_ref, wts_ref):
    b = pl.program_id(0)
    s = src_ref[0]
    sx = s[:, 0:1]
    sy = s[:, 1:2]
    sz = s[:, 2:3]
    cx = cp_ref[0:1, pl.ds(b * _K, _K)]
    cy = cp_ref[1:2, pl.ds(b * _K, _K)]
    cz = cp_ref[2:3, pl.ds(b * _K, _K)]
    dx = sx - cx
    sq = dx * dx
    dy = sy - cy
    sq = sq + dy * dy
    dz = sz - cz
    sq = sq + dz * dz
    logits = -sq / (2.0 * _SIGMA * _SIGMA)
    m = jnp.max(logits, axis=1, keepdims=True)
    e = jnp.exp(logits - m)
    wts = e / jnp.sum(e, axis=1, keepdims=True)
    wts_ref[0] = wts.astype(jnp.bfloat16)


def _rbf_wts(src, cp2):
    B, N, _ = src.shape
    return pl.pallas_call(
        _wts_kernel,
        grid=(B,),
        in_specs=[
            pl.BlockSpec((1, N, 3), lambda b: (b, 0, 0)),
            pl.BlockSpec(cp2.shape, lambda b: (0, 0)),
        ],
        out_specs=pl.BlockSpec((1, N, _K), lambda b: (b, 0, 0)),
        out_shape=jax.ShapeDtypeStruct((B, N, _K), jnp.bfloat16),
    )(src, cp2)


def _mlp_delta_kernel(xc_ref, xt_ref, w1_ref, b1_ref, g1_ref, be1_ref,
                      w2_ref, b2_ref, g2_ref, be2_ref, w3_ref, b3_ref,
                      src_ref, wts_ref, out_ref, delta_ref, w_ref, wsc):
    b = pl.program_id(0)

    @pl.when(b == 0)
    def _():
        F = xc_ref.shape[1]
        xc = xc_ref[...]
        xt = xt_ref[...]
        w1 = w1_ref[...]
        dn = (((1,), (1,)), ((), ()))
        h = jax.lax.dot_general(xc, w1[:, :F], dn,
                                preferred_element_type=jnp.float32)
        h = h + jax.lax.dot_general(xt, w1[:, F:], dn,
                                    preferred_element_type=jnp.float32)
        h = h + b1_ref[...][None, :]
        h = jnp.maximum(_bn(h, g1_ref[...], be1_ref[...]), 0.0)
        h = jax.lax.dot_general(h, w2_ref[...], dn,
                                preferred_element_type=jnp.float32)
        h = h + b2_ref[...][None, :]
        h = jnp.maximum(_bn(h, g2_ref[...], be2_ref[...]), 0.0)
        w = jax.lax.dot_general(h, w3_ref[...], dn,
                                preferred_element_type=jnp.float32)
        wsc[...] = w + b3_ref[...][None, :]

    wts = wts_ref[0].astype(jnp.float32)
    wb = wsc[pl.ds(b * _K, _K), :]
    w_ref[0] = wb
    delta3 = jax.lax.dot_general(wts, wb, (((1,), (0,)), ((), ())),
                                 preferred_element_type=jnp.float32)
    delta_ref[0] = delta3
    out_ref[0] = src_ref[0] + delta3


def _mlp_delta(xc, xt, W1, b1, g1, be1, W2, b2, g2, be2, W3, b3, src, wts):
    B, N, _ = src.shape
    R = xc.shape[0]
    F = xc.shape[1]
    H = W1.shape[0]
    P = W3.shape[0]
    cmap2 = lambda b: (0, 0)
    cmap1 = lambda b: (0,)
    return pl.pallas_call(
        _mlp_delta_kernel,
        grid=(B,),
        in_specs=[
            pl.BlockSpec((R, F), cmap2),
            pl.BlockSpec((R, F), cmap2),
            pl.BlockSpec((H, 2 * F), cmap2),
            pl.BlockSpec((H,), cmap1),
            pl.BlockSpec((H,), cmap1),
            pl.BlockSpec((H,), cmap1),
            pl.BlockSpec((H, H), cmap2),
            pl.BlockSpec((H,), cmap1),
            pl.BlockSpec((H,), cmap1),
            pl.BlockSpec((H,), cmap1),
            pl.BlockSpec((P, H), cmap2),
            pl.BlockSpec((P,), cmap1),
            pl.BlockSpec((1, N, 3), lambda b: (b, 0, 0)),
            pl.BlockSpec((1, N, _K), lambda b: (b, 0, 0)),
        ],
        out_specs=(
            pl.BlockSpec((1, N, 3), lambda b: (b, 0, 0)),
            pl.BlockSpec((1, N, 3), lambda b: (b, 0, 0)),
            pl.BlockSpec((1, _K, P), lambda b: (b, 0, 0)),
        ),
        out_shape=(
            jax.ShapeDtypeStruct((B, N, 3), jnp.float32),
            jax.ShapeDtypeStruct((B, N, 3), jnp.float32),
            jax.ShapeDtypeStruct((B, _K, P), jnp.float32),
        ),
        scratch_shapes=[pltpu.VMEM((R, P), jnp.float32)],
    )(xc, xt, W1, b1, g1, be1, W2, b2, g2, be2, W3, b3, src, wts)


def kernel(source_rigid_2, per_point_feats, tgt_points, tgt_point_feats,
           W1, b1, g1, be1, W2, b2, g2, be2, W3, b3):
    B, N, _ = source_rigid_2.shape
    M = tgt_points.shape[1]
    F = per_point_feats.shape[-1]

    src_t = jnp.transpose(source_rigid_2, (2, 0, 1))
    tgt_t = jnp.transpose(tgt_points, (2, 0, 1))

    cp_flat_idx, cp_t, nn_flat_idx = _fps_knn(src_t, tgt_t)

    ctrl_feats, tgt_feats = _sc_gather2(
        per_point_feats.reshape(B * N, F), cp_flat_idx.reshape(B * _K),
        tgt_point_feats.reshape(B * M, F), nn_flat_idx.reshape(B * _K))
    wts = _rbf_wts(source_rigid_2, cp_t.reshape(3, B * _K))

    output, delta, w = _mlp_delta(ctrl_feats, tgt_feats, W1, b1, g1, be1,
                                  W2, b2, g2, be2, W3, b3,
                                  source_rigid_2, wts)

    control_points = jnp.transpose(cp_t, (1, 2, 0))
    return (output, delta, control_points, w)


# speedup vs baseline: 1.0835x; 1.0835x over previous
import functools

import jax
import jax.numpy as jnp
from jax import lax
from jax.experimental import pallas as pl
from jax.experimental.pallas import tpu as pltpu
from jax.experimental.pallas import tpu_sc as plsc

_K = 128
_SIGMA = 0.5
_EPS = 1e-5
_H = 512


def _fps_kernel(pts_ref, tgt_ref, idx_ref, cp_ref, nn_ref):
    P = pts_ref[...]
    Bv = P.shape[0] // 3
    S, L = P.shape[1], P.shape[2]
    Nv = S * L
    n_id = (jax.lax.broadcasted_iota(jnp.int32, (Bv, S, L), 1) * L
            + jax.lax.broadcasted_iota(jnp.int32, (Bv, S, L), 2))
    kidx = jax.lax.broadcasted_iota(jnp.int32, (Bv, _K), 1)
    boff = jax.lax.broadcasted_iota(jnp.int32, (Bv, 1), 0) * Nv

    def step(k, carry):
        dist, far, idxs, cxa, cya, cza = carry
        oh = n_id == far
        ohb = jnp.concatenate([oh, oh, oh], axis=0)
        c = jnp.sum(jnp.where(ohb, P, 0.0), axis=(1, 2),
                    keepdims=True)
        sel = kidx == k
        idxs = jnp.where(sel, far.reshape(Bv, 1) + boff, idxs)
        cxa = jnp.where(sel, c[0 * Bv:1 * Bv].reshape(Bv, 1), cxa)
        cya = jnp.where(sel, c[1 * Bv:2 * Bv].reshape(Bv, 1), cya)
        cza = jnp.where(sel, c[2 * Bv:3 * Bv].reshape(Bv, 1), cza)
        D = P - c
        sq = D * D
        d = (sq[0 * Bv:1 * Bv] + sq[1 * Bv:2 * Bv]) + sq[2 * Bv:3 * Bv]
        dist = jnp.minimum(dist, d)
        m = jnp.max(dist, axis=(1, 2), keepdims=True)
        far = jnp.min(jnp.where(dist == m, n_id, Nv), axis=(1, 2),
                      keepdims=True).astype(jnp.int32)
        return dist, far, idxs, cxa, cya, cza

    dist0 = jnp.full((Bv, S, L), jnp.inf, jnp.float32)
    far0 = jnp.zeros((Bv, 1, 1), jnp.int32)
    z = jnp.zeros((Bv, _K), jnp.float32)
    carry = (dist0, far0, jnp.zeros((Bv, _K), jnp.int32), z, z, z)
    _, _, idxs, cxa, cya, cza = jax.lax.fori_loop(0, _K, step, carry,
                                                  unroll=4)
    idx_ref[...] = idxs
    cp_ref[0] = cxa
    cp_ref[1] = cya
    cp_ref[2] = cza

    Mv = tgt_ref.shape[2]
    lane = jax.lax.broadcasted_iota(jnp.int32, (_K, Mv), 1)
    for b in range(Bv):
        cx = cxa[b].reshape(_K, 1)
        cy = cya[b].reshape(_K, 1)
        cz = cza[b].reshape(_K, 1)
        tx = tgt_ref[0, b].reshape(1, Mv)
        ty = tgt_ref[1, b].reshape(1, Mv)
        tz = tgt_ref[2, b].reshape(1, Mv)
        dx = cx - tx
        d2 = dx * dx
        dy = cy - ty
        d2 = d2 + dy * dy
        dz = cz - tz
        d2 = d2 + dz * dz
        dk = jnp.sqrt(jnp.maximum(d2, 0.0))
        mk = jnp.min(dk, axis=1, keepdims=True)
        nn = jnp.min(jnp.where(dk == mk, lane, Mv), axis=1, keepdims=True)
        nn_ref[b, :] = (nn + b * Mv).reshape(_K).astype(jnp.int32)


def _fps_knn(src_t, tgt_t):
    B = src_t.shape[1]
    N = src_t.shape[2]
    S = 8
    pts = src_t.reshape(3 * B, S, N // S)
    return pl.pallas_call(
        _fps_kernel,
        out_shape=(
            jax.ShapeDtypeStruct((B, _K), jnp.int32),
            jax.ShapeDtypeStruct((3, B, _K), jnp.float32),
            jax.ShapeDtypeStruct((B, _K), jnp.int32),
        ),
    )(pts, tgt_t)


def _sc_gather2(tab1, idx1, tab2, idx2):
    R = idx1.shape[0]
    F = tab1.shape[1]
    info = plsc.get_sparse_core_info()
    nw = info.num_cores * info.num_subcores
    bpw = R // nw
    mesh = plsc.VectorSubcoreMesh(core_axis_name="c", subcore_axis_name="s")

    @functools.partial(
        pl.kernel,
        mesh=mesh,
        out_type=(
            jax.ShapeDtypeStruct((R, F), jnp.float32),
            jax.ShapeDtypeStruct((R, F), jnp.float32),
        ),
        scratch_types=[
            pltpu.VMEM((bpw,), jnp.int32),
            pltpu.VMEM((bpw, F), jnp.float32),
            pltpu.VMEM((bpw,), jnp.int32),
            pltpu.VMEM((bpw, F), jnp.float32),
            pltpu.SemaphoreType.DMA,
            pltpu.SemaphoreType.DMA,
        ],
    )
    def gather_k(t1, i1, t2, i2, o1, o2, iv1, rv1, iv2, rv2, s1, s2):
        wid = lax.axis_index("s") * info.num_cores + lax.axis_index("c")
        base = wid * bpw
        ci1 = pltpu.async_copy(i1.at[pl.ds(base, bpw)], iv1, s1)
        ci2 = pltpu.async_copy(i2.at[pl.ds(base, bpw)], iv2, s2)
        ci1.wait()
        c1 = pltpu.async_copy(t1.at[iv1], rv1, s1)
        ci2.wait()
        c2 = pltpu.async_copy(t2.at[iv2], rv2, s2)
        c1.wait()
        pltpu.sync_copy(rv1, o1.at[pl.ds(base, bpw)])
        c2.wait()
        pltpu.sync_copy(rv2, o2.at[pl.ds(base, bpw)])

    return gather_k(tab1, idx1, tab2, idx2)


def _bn(x, g, b):
    mu = jnp.mean(x, axis=0, keepdims=True)
    var = jnp.mean((x - mu) ** 2, axis=0, keepdims=True)
    return (x - mu) / jnp.sqrt(var + _EPS) * g[None, :] + b[None, :]


def _mlp_rbf_kernel(xc_ref, xt_ref, w1_ref, b1_ref, g1_ref, be1_ref,
                    w2_ref, b2_ref, g2_ref, be2_ref, w3_ref, b3_ref,
                    src_ref, cp_ref, out_ref, delta_ref, w_ref, wsc):
    b = pl.program_id(0)

    @pl.when(b == 0)
    def _():
        F = xc_ref.shape[1]
        xc = xc_ref[...]
        xt = xt_ref[...]
        w1 = w1_ref[...]
        dn = (((1,), (1,)), ((), ()))
        h = jax.lax.dot_general(xc, w1[:, :F], dn,
                                preferred_element_type=jnp.float32)
        h = h + jax.lax.dot_general(xt, w1[:, F:], dn,
                                    preferred_element_type=jnp.float32)
        h = h + b1_ref[...][None, :]
        h = jnp.maximum(_bn(h, g1_ref[...], be1_ref[...]), 0.0)
        h = jax.lax.dot_general(h, w2_ref[...], dn,
                                preferred_element_type=jnp.float32)
        h = h + b2_ref[...][None, :]
        h = jnp.maximum(_bn(h, g2_ref[...], be2_ref[...]), 0.0)
        w = jax.lax.dot_general(h, w3_ref[...], dn,
                                preferred_element_type=jnp.float32)
        wsc[...] = w + b3_ref[...][None, :]

    s = src_ref[0]
    sx = s[:, 0:1]
    sy = s[:, 1:2]
    sz = s[:, 2:3]
    cx = cp_ref[0:1, pl.ds(b * _K, _K)]
    cy = cp_ref[1:2, pl.ds(b * _K, _K)]
    cz = cp_ref[2:3, pl.ds(b * _K, _K)]
    dx = sx - cx
    sq = dx * dx
    dy = sy - cy
    sq = sq + dy * dy
    dz = sz - cz
    sq = sq + dz * dz
    logits = -sq / (2.0 * _SIGMA * _SIGMA)
    m = jnp.max(logits, axis=1, keepdims=True)
    e = jnp.exp(logits - m)
    wts = e / jnp.sum(e, axis=1, keepdims=True)
    wb = wsc[pl.ds(b * _K, _K), :]
    w_ref[0] = wb
    delta3 = jax.lax.dot_general(wts, wb, (((1,), (0,)), ((), ())),
                                 preferred_element_type=jnp.float32)
    delta_ref[0] = delta3
    out_ref[0] = s + delta3


def _mlp_rbf(xc, xt, W1, b1, g1, be1, W2, b2, g2, be2, W3, b3, src, cp2):
    B, N, _ = src.shape
    R = xc.shape[0]
    F = xc.shape[1]
    H = W1.shape[0]
    P = W3.shape[0]
    cmap2 = lambda b: (0, 0)
    cmap1 = lambda b: (0,)
    return pl.pallas_call(
        _mlp_rbf_kernel,
        grid=(B,),
        in_specs=[
            pl.BlockSpec((R, F), cmap2),
            pl.BlockSpec((R, F), cmap2),
            pl.BlockSpec((H, 2 * F), cmap2),
            pl.BlockSpec((H,), cmap1),
            pl.BlockSpec((H,), cmap1),
            pl.BlockSpec((H,), cmap1),
            pl.BlockSpec((H, H), cmap2),
            pl.BlockSpec((H,), cmap1),
            pl.BlockSpec((H,), cmap1),
            pl.BlockSpec((H,), cmap1),
            pl.BlockSpec((P, H), cmap2),
            pl.BlockSpec((P,), cmap1),
            pl.BlockSpec((1, N, 3), lambda b: (b, 0, 0)),
            pl.BlockSpec(cp2.shape, cmap2),
        ],
        out_specs=(
            pl.BlockSpec((1, N, 3), lambda b: (b, 0, 0)),
            pl.BlockSpec((1, N, 3), lambda b: (b, 0, 0)),
            pl.BlockSpec((1, _K, P), lambda b: (b, 0, 0)),
        ),
        out_shape=(
            jax.ShapeDtypeStruct((B, N, 3), jnp.float32),
            jax.ShapeDtypeStruct((B, N, 3), jnp.float32),
            jax.ShapeDtypeStruct((B, _K, P), jnp.float32),
        ),
        scratch_shapes=[pltpu.VMEM((R, P), jnp.float32)],
    )(xc, xt, W1, b1, g1, be1, W2, b2, g2, be2, W3, b3, src, cp2)


def kernel(source_rigid_2, per_point_feats, tgt_points, tgt_point_feats,
           W1, b1, g1, be1, W2, b2, g2, be2, W3, b3):
    B, N, _ = source_rigid_2.shape
    M = tgt_points.shape[1]
    F = per_point_feats.shape[-1]

    src_t = jnp.transpose(source_rigid_2, (2, 0, 1))
    tgt_t = jnp.transpose(tgt_points, (2, 0, 1))

    cp_flat_idx, cp_t, nn_flat_idx = _fps_knn(src_t, tgt_t)

    ctrl_feats, tgt_feats = _sc_gather2(
        per_point_feats.reshape(B * N, F), cp_flat_idx.reshape(B * _K),
        tgt_point_feats.reshape(B * M, F), nn_flat_idx.reshape(B * _K))

    output, delta, w = _mlp_rbf(ctrl_feats, tgt_feats, W1, b1, g1, be1,
                                W2, b2, g2, be2, W3, b3,
                                source_rigid_2, cp_t.reshape(3, B * _K))

    control_points = jnp.transpose(cp_t, (1, 2, 0))
    return (output, delta, control_points, w)


# speedup vs baseline: 1.0893x; 1.0054x over previous
import functools

import jax
import jax.numpy as jnp
from jax import lax
from jax.experimental import pallas as pl
from jax.experimental.pallas import tpu as pltpu
from jax.experimental.pallas import tpu_sc as plsc

_K = 128
_SIGMA = 0.5
_EPS = 1e-5


def _fps_kernel(pts_ref, tgt_ref, idx_ref, cp_ref, nn_ref):
    P = pts_ref[...]
    Bv = P.shape[0] // 3
    S, L = P.shape[1], P.shape[2]
    Nv = S * L
    n_id = (jax.lax.broadcasted_iota(jnp.int32, (Bv, S, L), 1) * L
            + jax.lax.broadcasted_iota(jnp.int32, (Bv, S, L), 2))
    kidx = jax.lax.broadcasted_iota(jnp.int32, (Bv, _K), 1)
    boff = jax.lax.broadcasted_iota(jnp.int32, (Bv, 1), 0) * Nv

    def step(k, carry):
        dist, far, idxs, cxa, cya, cza = carry
        oh = n_id == far
        ohb = jnp.concatenate([oh, oh, oh], axis=0)
        c = jnp.sum(jnp.where(ohb, P, 0.0), axis=(1, 2),
                    keepdims=True)
        sel = kidx == k
        idxs = jnp.where(sel, far.reshape(Bv, 1) + boff, idxs)
        cxa = jnp.where(sel, c[0 * Bv:1 * Bv].reshape(Bv, 1), cxa)
        cya = jnp.where(sel, c[1 * Bv:2 * Bv].reshape(Bv, 1), cya)
        cza = jnp.where(sel, c[2 * Bv:3 * Bv].reshape(Bv, 1), cza)
        D = P - c
        sq = D * D
        d = (sq[0 * Bv:1 * Bv] + sq[1 * Bv:2 * Bv]) + sq[2 * Bv:3 * Bv]
        dist = jnp.minimum(dist, d)
        m = jnp.max(dist, axis=(1, 2), keepdims=True)
        far = jnp.min(jnp.where(dist == m, n_id, Nv), axis=(1, 2),
                      keepdims=True).astype(jnp.int32)
        return dist, far, idxs, cxa, cya, cza

    dist0 = jnp.full((Bv, S, L), jnp.inf, jnp.float32)
    far0 = jnp.zeros((Bv, 1, 1), jnp.int32)
    z = jnp.zeros((Bv, _K), jnp.float32)
    carry = (dist0, far0, jnp.zeros((Bv, _K), jnp.int32), z, z, z)
    _, _, idxs, cxa, cya, cza = jax.lax.fori_loop(0, _K, step, carry,
                                                  unroll=4)
    idx_ref[...] = idxs
    cp_ref[0] = cxa
    cp_ref[1] = cya
    cp_ref[2] = cza

    Mv = tgt_ref.shape[2]
    lane = jax.lax.broadcasted_iota(jnp.int32, (_K, Mv), 1)
    for b in range(Bv):
        cx = cxa[b].reshape(_K, 1)
        cy = cya[b].reshape(_K, 1)
        cz = cza[b].reshape(_K, 1)
        tx = tgt_ref[0, b].reshape(1, Mv)
        ty = tgt_ref[1, b].reshape(1, Mv)
        tz = tgt_ref[2, b].reshape(1, Mv)
        dx = cx - tx
        d2 = dx * dx
        dy = cy - ty
        d2 = d2 + dy * dy
        dz = cz - tz
        d2 = d2 + dz * dz
        dk = jnp.sqrt(jnp.maximum(d2, 0.0))
        mk = jnp.min(dk, axis=1, keepdims=True)
        nn = jnp.min(jnp.where(dk == mk, lane, Mv), axis=1, keepdims=True)
        nn_ref[b, :] = (nn + b * Mv).reshape(_K).astype(jnp.int32)


def _fps_knn(src_t, tgt_t):
    B = src_t.shape[1]
    N = src_t.shape[2]
    S = 8
    pts = src_t.reshape(3 * B, S, N // S)
    return pl.pallas_call(
        _fps_kernel,
        out_shape=(
            jax.ShapeDtypeStruct((B, _K), jnp.int32),
            jax.ShapeDtypeStruct((3, B, _K), jnp.float32),
            jax.ShapeDtypeStruct((B, _K), jnp.int32),
        ),
    )(pts, tgt_t)


def _sc_gather2(tab1, idx1, tab2, idx2):
    R = idx1.shape[0]
    F = tab1.shape[1]
    info = plsc.get_sparse_core_info()
    nw = info.num_cores * info.num_subcores
    bpw = R // nw
    mesh = plsc.VectorSubcoreMesh(core_axis_name="c", subcore_axis_name="s")

    @functools.partial(
        pl.kernel,
        mesh=mesh,
        out_type=(
            jax.ShapeDtypeStruct((R, F), jnp.float32),
            jax.ShapeDtypeStruct((R, F), jnp.float32),
        ),
        scratch_types=[
            pltpu.VMEM((bpw,), jnp.int32),
            pltpu.VMEM((bpw, F), jnp.float32),
            pltpu.VMEM((bpw,), jnp.int32),
            pltpu.VMEM((bpw, F), jnp.float32),
            pltpu.SemaphoreType.DMA,
            pltpu.SemaphoreType.DMA,
        ],
    )
    def gather_k(t1, i1, t2, i2, o1, o2, iv1, rv1, iv2, rv2, s1, s2):
        wid = lax.axis_index("s") * info.num_cores + lax.axis_index("c")
        base = wid * bpw
        ci1 = pltpu.async_copy(i1.at[pl.ds(base, bpw)], iv1, s1)
        ci2 = pltpu.async_copy(i2.at[pl.ds(base, bpw)], iv2, s2)
        ci1.wait()
        c1 = pltpu.async_copy(t1.at[iv1], rv1, s1)
        ci2.wait()
        c2 = pltpu.async_copy(t2.at[iv2], rv2, s2)
        c1.wait()
        pltpu.sync_copy(rv1, o1.at[pl.ds(base, bpw)])
        c2.wait()
        pltpu.sync_copy(rv2, o2.at[pl.ds(base, bpw)])

    return gather_k(tab1, idx1, tab2, idx2)


def _bn(x, g, b):
    mu = jnp.mean(x, axis=0, keepdims=True)
    var = jnp.mean((x - mu) ** 2, axis=0, keepdims=True)
    return (x - mu) / jnp.sqrt(var + _EPS) * g[None, :] + b[None, :]


def _mlp_rbf_kernel(xc_ref, xt_ref, w1_ref, b1_ref, g1_ref, be1_ref,
                    w2_ref, b2_ref, g2_ref, be2_ref, w3_ref, b3_ref,
                    src_ref, cp_ref, out_ref, delta_ref, w_ref, wsc):
    b = pl.program_id(0)

    @pl.when(b == 0)
    def _():
        F = xc_ref.shape[1]
        xc = xc_ref[...]
        xt = xt_ref[...]
        w1 = w1_ref[...]
        dn = (((1,), (1,)), ((), ()))
        h = jax.lax.dot_general(xc, w1[:, :F], dn,
                                preferred_element_type=jnp.float32)
        h = h + jax.lax.dot_general(xt, w1[:, F:], dn,
                                    preferred_element_type=jnp.float32)
        h = h + b1_ref[...][None, :]
        h = jnp.maximum(_bn(h, g1_ref[...], be1_ref[...]), 0.0)
        h = jax.lax.dot_general(h, w2_ref[...], dn,
                                preferred_element_type=jnp.float32)
        h = h + b2_ref[...][None, :]
        h = jnp.maximum(_bn(h, g2_ref[...], be2_ref[...]), 0.0)
        w = jax.lax.dot_general(h, w3_ref[...], dn,
                                preferred_element_type=jnp.float32)
        wsc[...] = w + b3_ref[...][None, :]

    s = src_ref[0]
    sx = s[:, 0:1]
    sy = s[:, 1:2]
    sz = s[:, 2:3]
    cx = cp_ref[0:1, pl.ds(b * _K, _K)]
    cy = cp_ref[1:2, pl.ds(b * _K, _K)]
    cz = cp_ref[2:3, pl.ds(b * _K, _K)]
    dx = sx - cx
    sq = dx * dx
    dy = sy - cy
    sq = sq + dy * dy
    dz = sz - cz
    sq = sq + dz * dz
    logits = -sq / (2.0 * _SIGMA * _SIGMA)
    m = jnp.max(logits, axis=1, keepdims=True)
    e = jnp.exp(logits - m)
    wts = e / jnp.sum(e, axis=1, keepdims=True)
    wb = wsc[pl.ds(b * _K, _K), :]
    w_ref[0] = wb
    delta3 = jax.lax.dot_general(wts, wb, (((1,), (0,)), ((), ())),
                                 preferred_element_type=jnp.float32)
    delta_ref[0] = delta3
    out_ref[0] = s + delta3


def _mlp_rbf(xc, xt, W1, b1, g1, be1, W2, b2, g2, be2, W3, b3, src, cp2):
    B, N, _ = src.shape
    R = xc.shape[0]
    F = xc.shape[1]
    H = W1.shape[0]
    P = W3.shape[0]
    cmap2 = lambda b: (0, 0)
    cmap1 = lambda b: (0,)
    return pl.pallas_call(
        _mlp_rbf_kernel,
        grid=(B,),
        in_specs=[
            pl.BlockSpec((R, F), cmap2),
            pl.BlockSpec((R, F), cmap2),
            pl.BlockSpec((H, 2 * F), cmap2),
            pl.BlockSpec((H,), cmap1),
            pl.BlockSpec((H,), cmap1),
            pl.BlockSpec((H,), cmap1),
            pl.BlockSpec((H, H), cmap2),
            pl.BlockSpec((H,), cmap1),
            pl.BlockSpec((H,), cmap1),
            pl.BlockSpec((H,), cmap1),
            pl.BlockSpec((P, H), cmap2),
            pl.BlockSpec((P,), cmap1),
            pl.BlockSpec((1, N, 3), lambda b: (b, 0, 0)),
            pl.BlockSpec(cp2.shape, cmap2),
        ],
        out_specs=(
            pl.BlockSpec((1, N, 3), lambda b: (b, 0, 0)),
            pl.BlockSpec((1, N, 3), lambda b: (b, 0, 0)),
            pl.BlockSpec((1, _K, P), lambda b: (b, 0, 0)),
        ),
        out_shape=(
            jax.ShapeDtypeStruct((B, N, 3), jnp.float32),
            jax.ShapeDtypeStruct((B, N, 3), jnp.float32),
            jax.ShapeDtypeStruct((B, _K, P), jnp.float32),
        ),
        scratch_shapes=[pltpu.VMEM((R, P), jnp.float32)],
    )(xc, xt, W1, b1, g1, be1, W2, b2, g2, be2, W3, b3, src, cp2)


def kernel(source_rigid_2, per_point_feats, tgt_points, tgt_point_feats,
           W1, b1, g1, be1, W2, b2, g2, be2, W3, b3):
    B, N, _ = source_rigid_2.shape
    M = tgt_points.shape[1]
    F = per_point_feats.shape[-1]

    src_t = jnp.transpose(source_rigid_2, (2, 0, 1))
    tgt_t = jnp.transpose(tgt_points, (2, 0, 1))

    cp_flat_idx, cp_t, nn_flat_idx = _fps_knn(src_t, tgt_t)

    ctrl_feats, tgt_feats = _sc_gather2(
        per_point_feats.reshape(B * N, F), cp_flat_idx.reshape(B * _K),
        tgt_point_feats.reshape(B * M, F), nn_flat_idx.reshape(B * _K))

    output, delta, w = _mlp_rbf(ctrl_feats, tgt_feats, W1, b1, g1, be1,
                                W2, b2, g2, be2, W3, b3,
                                source_rigid_2, cp_t.reshape(3, B * _K))

    control_points = jnp.transpose(cp_t, (1, 2, 0))
    return (output, delta, control_points, w)
